# Initial kernel scaffold; baseline (speedup 1.0000x reference)
import jax
import jax.numpy as jnp
from jax.experimental import pallas as pl


def kernel(action_mask):
    raise NotImplementedError("write your pallas kernel here")


# speedup vs baseline: 1.3369x; 1.3369x over previous
import jax
import jax.numpy as jnp
from jax import lax
from jax.experimental import pallas as pl
from jax.experimental.pallas import tpu as pltpu

_B = 128
_N = 32768
_BN = 2048
_NBLK = _N // _BN


def _body(mask_ref, out_ref, seen_ref):
    j = pl.program_id(0)

    @pl.when(j == 0)
    def _init():
        seen_ref[...] = jnp.zeros_like(seen_ref)

    m = mask_ref[...]
    iota = lax.broadcasted_iota(jnp.int32, (_B, _BN), 1)
    idx = jnp.where(m, iota, _BN)
    local_min = jnp.min(idx, axis=1, keepdims=True)
    seen = seen_ref[:, :1]
    hit = (iota == local_min) & (seen == 0)
    out_ref[...] = jnp.where(hit, jnp.float32(10.0), jnp.float32(0.0))
    local_any = (local_min < _BN).astype(jnp.int32)
    seen_ref[...] = seen_ref[...] | jnp.broadcast_to(local_any, (_B, _B))


def kernel(action_mask):
    return pl.pallas_call(
        _body,
        grid=(_NBLK,),
        in_specs=[pl.BlockSpec((_B, _BN), lambda j: (0, j))],
        out_specs=pl.BlockSpec((_B, _BN), lambda j: (0, j)),
        out_shape=jax.ShapeDtypeStruct((_B, _N), jnp.float32),
        scratch_shapes=[pltpu.VMEM((_B, _B), jnp.int32)],
        compiler_params=pltpu.CompilerParams(
            dimension_semantics=("arbitrary",),
        ),
    )(action_mask)
